# Initial kernel scaffold; baseline (speedup 1.0000x reference)
import functools

import jax
import jax.numpy as jnp
from jax import lax
from jax.experimental import pallas as pl
from jax.experimental.pallas import tpu as pltpu
from jax.experimental.pallas import tpu_sc as plsc

DIM = 32
NW = 32
CH = 3200


@functools.partial(jax.jit, static_argnames=("b_total",))
def _sc_gather(ids_flat, table, b_total):
    b_per_w = b_total // NW
    n_chunks = b_per_w // CH
    mesh = plsc.VectorSubcoreMesh(core_axis_name="c", subcore_axis_name="s")

    @functools.partial(
        pl.kernel,
        mesh=mesh,
        out_type=jax.ShapeDtypeStruct((b_total, DIM), jnp.float32),
        scratch_types=[
            pltpu.VMEM((CH,), jnp.int32),
            pltpu.VMEM((CH, DIM), jnp.float32),
            pltpu.SemaphoreType.DMA,
        ],
    )
    def k(ids_hbm, table_hbm, out_hbm, idx_v, rows_v, sem):
        wid = lax.axis_index("s") * 2 + lax.axis_index("c")
        base_w = wid * b_per_w

        def body(i, carry):
            base = base_w + i * CH
            pltpu.sync_copy(ids_hbm.at[pl.ds(base, CH)], idx_v)
            pltpu.async_copy(table_hbm.at[idx_v], rows_v, sem).wait()
            pltpu.sync_copy(rows_v, out_hbm.at[pl.ds(base, CH)])
            return carry

        lax.fori_loop(0, n_chunks, body, 0)

    return k(ids_flat, table)


def kernel(ids, table):
    b, h = ids.shape
    ids_flat = ids.reshape(-1).astype(jnp.int32)
    out = _sc_gather(ids_flat, table, b * h)
    return out.reshape(b, h, DIM)


# speedup vs baseline: 1.1101x; 1.1101x over previous
import functools

import jax
import jax.numpy as jnp
from jax import lax
from jax.experimental import pallas as pl
from jax.experimental.pallas import tpu as pltpu
from jax.experimental.pallas import tpu_sc as plsc

DIM = 32
NW = 32
CH = 3200


@functools.partial(jax.jit, static_argnames=("b_total",))
def _sc_gather(ids_flat, table, b_total):
    b_per_w = b_total // NW
    n_chunks = b_per_w // CH
    mesh = plsc.VectorSubcoreMesh(core_axis_name="c", subcore_axis_name="s")

    @functools.partial(
        pl.kernel,
        mesh=mesh,
        out_type=jax.ShapeDtypeStruct((b_total, DIM), jnp.float32),
        scratch_types=[
            pltpu.VMEM((CH,), jnp.int32),
            pltpu.VMEM((CH, DIM), jnp.float32),
            pltpu.SemaphoreType.DMA,
        ],
        compiler_params=pltpu.CompilerParams(use_tc_tiling_on_sc=False),
    )
    def k(ids_hbm, table_hbm, out_hbm, idx_v, rows_v, sem):
        wid = lax.axis_index("s") * 2 + lax.axis_index("c")
        base_w = wid * b_per_w

        def body(i, carry):
            base = base_w + i * CH
            pltpu.sync_copy(ids_hbm.at[pl.ds(base, CH)], idx_v)
            pltpu.async_copy(table_hbm.at[idx_v], rows_v, sem).wait()
            pltpu.sync_copy(rows_v, out_hbm.at[pl.ds(base, CH)])
            return carry

        lax.fori_loop(0, n_chunks, body, 0)

    return k(ids_flat, table)


def kernel(ids, table):
    b, h = ids.shape
    ids_flat = ids.reshape(-1).astype(jnp.int32)
    out = _sc_gather(ids_flat, table, b * h)
    return out.reshape(b, h, DIM)


# speedup vs baseline: 1.1122x; 1.0019x over previous
import functools

import jax
import jax.numpy as jnp
from jax import lax
from jax.experimental import pallas as pl
from jax.experimental.pallas import tpu as pltpu
from jax.experimental.pallas import tpu_sc as plsc

DIM = 32
NW = 32
CH = 1600


@functools.partial(jax.jit, static_argnames=("b_total",))
def _sc_gather(ids_flat, table, b_total):
    b_per_w = b_total // NW
    n_chunks = b_per_w // CH
    mesh = plsc.VectorSubcoreMesh(core_axis_name="c", subcore_axis_name="s")

    @functools.partial(
        pl.kernel,
        mesh=mesh,
        out_type=jax.ShapeDtypeStruct((b_total, DIM), jnp.float32),
        scratch_types=[
            pltpu.VMEM((CH,), jnp.int32),
            pltpu.VMEM((CH,), jnp.int32),
            pltpu.VMEM((CH, DIM), jnp.float32),
            pltpu.VMEM((CH, DIM), jnp.float32),
            pltpu.SemaphoreType.DMA,
            pltpu.SemaphoreType.DMA,
            pltpu.SemaphoreType.DMA,
            pltpu.SemaphoreType.DMA,
        ],
        compiler_params=pltpu.CompilerParams(use_tc_tiling_on_sc=False),
    )
    def k(ids_hbm, table_hbm, out_hbm, idx0, idx1, rows0, rows1,
          g0, g1, s0, s1):
        idx = (idx0, idx1)
        rows = (rows0, rows1)
        gsem = (g0, g1)
        ssem = (s0, s1)
        wid = lax.axis_index("s") * 2 + lax.axis_index("c")
        base_w = wid * b_per_w

        pltpu.sync_copy(ids_hbm.at[pl.ds(base_w, CH)], idx[0])
        g_copy = [None, None]
        s_copy = [None, None]
        g_copy[0] = pltpu.async_copy(table_hbm.at[idx[0]], rows[0], gsem[0])

        for i in range(n_chunks):
            cur = i % 2
            nxt = 1 - cur
            if i + 1 < n_chunks:
                base_n = base_w + (i + 1) * CH
                pltpu.sync_copy(ids_hbm.at[pl.ds(base_n, CH)], idx[nxt])
                if i >= 1:
                    s_copy[nxt].wait()
                g_copy[nxt] = pltpu.async_copy(
                    table_hbm.at[idx[nxt]], rows[nxt], gsem[nxt])
            g_copy[cur].wait()
            s_copy[cur] = pltpu.async_copy(
                rows[cur], out_hbm.at[pl.ds(base_w + i * CH, CH)], ssem[cur])

        s_copy[(n_chunks - 2) % 2].wait()
        s_copy[(n_chunks - 1) % 2].wait()

    return k(ids_flat, table)


def kernel(ids, table):
    b, h = ids.shape
    ids_flat = ids.reshape(-1).astype(jnp.int32)
    out = _sc_gather(ids_flat, table, b * h)
    return out.reshape(b, h, DIM)
